# Initial kernel scaffold; baseline (speedup 1.0000x reference)
import jax
import jax.numpy as jnp
from jax.experimental import pallas as pl


def kernel(x, edge_index, W1, b1, W2, b2):
    raise NotImplementedError("write your pallas kernel here")


# speedup vs baseline: 12.1184x; 12.1184x over previous
import functools

import jax
import jax.numpy as jnp
from jax import lax
from jax.experimental import pallas as pl
from jax.experimental.pallas import tpu as pltpu
from jax.experimental.pallas import tpu_sc as plsc

NC = 2
NS = 16
NW = NC * NS
L = 16
CHUNK = 80


def _sc_mesh():
    return plsc.VectorSubcoreMesh(
        core_axis_name="c", subcore_axis_name="s", num_cores=NC, num_subcores=NS
    )


def _zero_rows(buf, rows, cols):
    zero = jnp.zeros((L,), jnp.float32)

    def body(r, _):
        for c in range(cols // L):
            buf[r, pl.ds(c * L, L)] = zero
        return 0

    lax.fori_loop(0, rows, body, 0, unroll=False)


def _fill_ones(buf, rows, cols):
    one = jnp.full((L,), 1.0, jnp.float32)

    def body(r, _):
        for c in range(cols // L):
            buf[r, pl.ds(c * L, L)] = one
        return 0

    lax.fori_loop(0, rows, body, 0, unroll=False)


def _make_agg_kernel(n_pad, e, d, count_only=False):
    assert e % (NW * CHUNK) == 0 and n_pad % (NS * 8) == 0
    iters = e // (NW * CHUNK)
    rows_per_tile = n_pad // NS
    assert rows_per_tile % 5 == 0 and (rows_per_tile // 5) % 8 == 0
    copy_rows = rows_per_tile // 5

    def agg_body(g_hbm, src_hbm, dst_hbm, out_hbm,
                 src_v, dst_v, rows_v, bounce_v, acc_sh, sem):
        cid = lax.axis_index("c")
        sid = lax.axis_index("s")
        wid = sid * NC + cid
        row0 = sid * rows_per_tile
        _zero_rows(bounce_v, copy_rows, d)
        if count_only:
            _fill_ones(rows_v, CHUNK, d)
        for j in range(5):
            pltpu.sync_copy(bounce_v, acc_sh.at[pl.ds(row0 + j * copy_rows, copy_rows)])
        plsc.subcore_barrier()

        base = wid * (iters * CHUNK)

        def body(i, _):
            off = base + i * CHUNK
            pltpu.sync_copy(dst_hbm.at[pl.ds(off, CHUNK)], dst_v)
            if not count_only:
                pltpu.sync_copy(src_hbm.at[pl.ds(off, CHUNK)], src_v)
                pltpu.async_copy(g_hbm.at[src_v], rows_v, sem).wait()
            pltpu.sync_copy(rows_v, acc_sh.at[dst_v], add=True)
            return 0

        lax.fori_loop(0, iters, body, 0, unroll=False)
        plsc.subcore_barrier()
        for j in range(5):
            r = row0 + j * copy_rows
            pltpu.sync_copy(acc_sh.at[pl.ds(r, copy_rows)], bounce_v)
            pltpu.sync_copy(bounce_v, out_hbm.at[cid, pl.ds(r, copy_rows)])

    kernel_kwargs = dict(
        out_type=jax.ShapeDtypeStruct((NC, n_pad, d), jnp.float32),
        mesh=_sc_mesh(),
        scratch_types=[
            pltpu.VMEM((CHUNK,), jnp.int32),
            pltpu.VMEM((CHUNK,), jnp.int32),
            pltpu.VMEM((CHUNK, d), jnp.float32),
            pltpu.VMEM((copy_rows, d), jnp.float32),
            pltpu.VMEM_SHARED((n_pad, d), jnp.float32),
            pltpu.SemaphoreType.DMA,
        ],
    )
    if count_only:
        @functools.partial(pl.kernel, **kernel_kwargs)
        def count_kernel(dst_hbm, out_hbm, src_v, dst_v, rows_v, bounce_v,
                         acc_sh, sem):
            agg_body(None, None, dst_hbm, out_hbm,
                     src_v, dst_v, rows_v, bounce_v, acc_sh, sem)

        return count_kernel

    @functools.partial(pl.kernel, **kernel_kwargs)
    def agg_kernel(g_hbm, src_hbm, dst_hbm, out_hbm,
                   src_v, dst_v, rows_v, bounce_v, acc_sh, sem):
        agg_body(g_hbm, src_hbm, dst_hbm, out_hbm,
                 src_v, dst_v, rows_v, bounce_v, acc_sh, sem)

    return agg_kernel


_BLK = 400


def _tc1_body(x_ref, w_ref, deg_ref, g_ref, dis_ref):
    d = deg_ref[0][:, :1] + deg_ref[1][:, :1] + 1.0
    di = lax.rsqrt(d)
    dis_ref[...] = di + jnp.zeros((_BLK, L), jnp.float32)
    m = jnp.dot(x_ref[...], w_ref[...], preferred_element_type=jnp.float32)
    g_ref[...] = m * di


def _tc2_body(acc_ref, g_ref, dis_ref, b_ref, w_ref, out_ref):
    di = dis_ref[...][:, :1]
    pre = (acc_ref[0] + acc_ref[1] + g_ref[...]) * di + b_ref[...]
    h = jnp.maximum(pre, 0.0)
    out_ref[...] = jnp.dot(h, w_ref[...], preferred_element_type=jnp.float32) * di


def _tc3_body(acc_ref, g_ref, dis_ref, b_ref, out_ref):
    di = dis_ref[...][:, :1]
    pre = (acc_ref[0] + acc_ref[1] + g_ref[...]) * di + b_ref[...]
    out_ref[...] = jnp.maximum(pre, 0.0)


def _row_spec(d):
    return pl.BlockSpec((_BLK, d), lambda i: (i, 0))


def _pair_spec(d):
    return pl.BlockSpec((NC, _BLK, d), lambda i: (0, i, 0))


def _full_spec(shape):
    return pl.BlockSpec(shape, lambda i: tuple(0 for _ in shape))


def kernel(x, edge_index, W1, b1, W2, b2):
    n, d0 = x.shape
    h1 = W1.shape[1]
    h2 = W2.shape[1]
    e = edge_index.shape[1]
    src = edge_index[0].astype(jnp.int32)
    dst = edge_index[1].astype(jnp.int32)
    b1m = b1.reshape(1, h1)
    b2m = b2.reshape(1, h2)
    n_pad = ((n + NS * 8 - 1) // (NS * 8)) * (NS * 8)
    if (n_pad // NS) % 40 != 0:
        n_pad = ((n + NS * 40 - 1) // (NS * 40)) * (NS * 40)

    deg = _make_agg_kernel(n_pad, e, h1, count_only=True)(dst)

    grid = (n // _BLK,)
    g1, dis = pl.pallas_call(
        _tc1_body,
        grid=grid,
        in_specs=[_row_spec(d0), _full_spec((d0, h1)), _pair_spec(h1)],
        out_specs=[_row_spec(h1), _row_spec(L)],
        out_shape=[
            jax.ShapeDtypeStruct((n, h1), jnp.float32),
            jax.ShapeDtypeStruct((n, L), jnp.float32),
        ],
    )(x, W1, deg)

    agg = _make_agg_kernel(n_pad, e, h1)
    acc1 = agg(g1, src, dst)

    g2 = pl.pallas_call(
        _tc2_body,
        grid=grid,
        in_specs=[_pair_spec(h1), _row_spec(h1), _row_spec(L),
                  _full_spec((1, h1)), _full_spec((h1, h2))],
        out_specs=_row_spec(h2),
        out_shape=jax.ShapeDtypeStruct((n, h2), jnp.float32),
    )(acc1, g1, dis, b1m, W2)

    acc2 = agg(g2, src, dst)

    out = pl.pallas_call(
        _tc3_body,
        grid=grid,
        in_specs=[_pair_spec(h2), _row_spec(h2), _row_spec(L),
                  _full_spec((1, h2))],
        out_specs=_row_spec(h2),
        out_shape=jax.ShapeDtypeStruct((n, h2), jnp.float32),
    )(acc2, g2, dis, b2m)

    return out


# speedup vs baseline: 13.8483x; 1.1428x over previous
import functools

import jax
import jax.numpy as jnp
from jax import lax
from jax.experimental import pallas as pl
from jax.experimental.pallas import tpu as pltpu
from jax.experimental.pallas import tpu_sc as plsc

NC = 2
NS = 16
NW = NC * NS
L = 16
CHUNK = 80


def _sc_mesh():
    return plsc.VectorSubcoreMesh(
        core_axis_name="c", subcore_axis_name="s", num_cores=NC, num_subcores=NS
    )


def _zero_rows(buf, rows, cols):
    zero = jnp.zeros((L,), jnp.float32)

    def body(r, _):
        for c in range(cols // L):
            buf[r, pl.ds(c * L, L)] = zero
        return 0

    lax.fori_loop(0, rows, body, 0, unroll=False)


def _fill_ones(buf, rows, cols):
    one = jnp.full((L,), 1.0, jnp.float32)

    def body(r, _):
        for c in range(cols // L):
            buf[r, pl.ds(c * L, L)] = one
        return 0

    lax.fori_loop(0, rows, body, 0, unroll=False)


def _make_agg_kernel(n_pad, e, d, count_only=False):
    assert e % (NW * CHUNK) == 0 and n_pad % (NS * 8) == 0
    iters = e // (NW * CHUNK)
    rows_per_tile = n_pad // NS
    assert rows_per_tile % 5 == 0 and (rows_per_tile // 5) % 8 == 0
    copy_rows = rows_per_tile // 5

    def agg_body(g_hbm, src_hbm, dst_hbm, out_hbm,
                 src_v, dst_v, rows_v, bounce_v, acc_sh, sem):
        cid = lax.axis_index("c")
        sid = lax.axis_index("s")
        wid = sid * NC + cid
        row0 = sid * rows_per_tile
        _zero_rows(bounce_v, copy_rows, d)
        if count_only:
            _fill_ones(rows_v, CHUNK, d)
        for j in range(5):
            pltpu.sync_copy(bounce_v, acc_sh.at[pl.ds(row0 + j * copy_rows, copy_rows)])
        plsc.subcore_barrier()

        base = wid * (iters * CHUNK)
        if not count_only:
            pltpu.sync_copy(src_hbm.at[pl.ds(base, iters * CHUNK)], src_v)

        def body(i, _):
            off = base + i * CHUNK
            pltpu.sync_copy(dst_hbm.at[pl.ds(off, CHUNK)], dst_v)
            if not count_only:
                pltpu.async_copy(g_hbm.at[src_v.at[pl.ds(i * CHUNK, CHUNK)]],
                                 rows_v, sem).wait()
            pltpu.sync_copy(rows_v, acc_sh.at[dst_v], add=True)
            return 0

        lax.fori_loop(0, iters, body, 0, unroll=False)
        plsc.subcore_barrier()
        for j in range(5):
            r = row0 + j * copy_rows
            pltpu.sync_copy(acc_sh.at[pl.ds(r, copy_rows)], bounce_v)
            pltpu.sync_copy(bounce_v, out_hbm.at[cid, pl.ds(r, copy_rows)])

    kernel_kwargs = dict(
        out_type=jax.ShapeDtypeStruct((NC, n_pad, d), jnp.float32),
        mesh=_sc_mesh(),
        scratch_types=[
            pltpu.VMEM((e // NW,), jnp.int32),
            pltpu.VMEM((CHUNK,), jnp.int32),
            pltpu.VMEM((CHUNK, d), jnp.float32),
            pltpu.VMEM((copy_rows, d), jnp.float32),
            pltpu.VMEM_SHARED((n_pad, d), jnp.float32),
            pltpu.SemaphoreType.DMA,
        ],
    )
    if count_only:
        @functools.partial(pl.kernel, **kernel_kwargs)
        def count_kernel(dst_hbm, out_hbm, src_v, dst_v, rows_v, bounce_v,
                         acc_sh, sem):
            agg_body(None, None, dst_hbm, out_hbm,
                     src_v, dst_v, rows_v, bounce_v, acc_sh, sem)

        return count_kernel

    @functools.partial(pl.kernel, **kernel_kwargs)
    def agg_kernel(g_hbm, src_hbm, dst_hbm, out_hbm,
                   src_v, dst_v, rows_v, bounce_v, acc_sh, sem):
        agg_body(g_hbm, src_hbm, dst_hbm, out_hbm,
                 src_v, dst_v, rows_v, bounce_v, acc_sh, sem)

    return agg_kernel


_BLK = 400


def _tc1_body(x_ref, w_ref, deg_ref, g_ref, dis_ref):
    d = deg_ref[0][:, :1] + deg_ref[1][:, :1] + 1.0
    di = lax.rsqrt(d)
    dis_ref[...] = di + jnp.zeros((_BLK, L), jnp.float32)
    m = jnp.dot(x_ref[...], w_ref[...], preferred_element_type=jnp.float32)
    g_ref[...] = m * di


def _tc2_body(acc_ref, g_ref, dis_ref, b_ref, w_ref, out_ref):
    di = dis_ref[...][:, :1]
    pre = (acc_ref[0] + acc_ref[1] + g_ref[...]) * di + b_ref[...]
    h = jnp.maximum(pre, 0.0)
    out_ref[...] = jnp.dot(h, w_ref[...], preferred_element_type=jnp.float32) * di


def _tc3_body(acc_ref, g_ref, dis_ref, b_ref, out_ref):
    di = dis_ref[...][:, :1]
    pre = (acc_ref[0] + acc_ref[1] + g_ref[...]) * di + b_ref[...]
    out_ref[...] = jnp.maximum(pre, 0.0)


def _row_spec(d):
    return pl.BlockSpec((_BLK, d), lambda i: (i, 0))


def _pair_spec(d):
    return pl.BlockSpec((NC, _BLK, d), lambda i: (0, i, 0))


def _full_spec(shape):
    return pl.BlockSpec(shape, lambda i: tuple(0 for _ in shape))


def kernel(x, edge_index, W1, b1, W2, b2):
    n, d0 = x.shape
    h1 = W1.shape[1]
    h2 = W2.shape[1]
    e = edge_index.shape[1]
    src = edge_index[0].astype(jnp.int32)
    dst = edge_index[1].astype(jnp.int32)
    b1m = b1.reshape(1, h1)
    b2m = b2.reshape(1, h2)
    n_pad = ((n + NS * 8 - 1) // (NS * 8)) * (NS * 8)
    if (n_pad // NS) % 40 != 0:
        n_pad = ((n + NS * 40 - 1) // (NS * 40)) * (NS * 40)

    deg = _make_agg_kernel(n_pad, e, h1, count_only=True)(dst)

    grid = (n // _BLK,)
    g1, dis = pl.pallas_call(
        _tc1_body,
        grid=grid,
        in_specs=[_row_spec(d0), _full_spec((d0, h1)), _pair_spec(h1)],
        out_specs=[_row_spec(h1), _row_spec(L)],
        out_shape=[
            jax.ShapeDtypeStruct((n, h1), jnp.float32),
            jax.ShapeDtypeStruct((n, L), jnp.float32),
        ],
    )(x, W1, deg)

    agg = _make_agg_kernel(n_pad, e, h1)
    acc1 = agg(g1, src, dst)

    g2 = pl.pallas_call(
        _tc2_body,
        grid=grid,
        in_specs=[_pair_spec(h1), _row_spec(h1), _row_spec(L),
                  _full_spec((1, h1)), _full_spec((h1, h2))],
        out_specs=_row_spec(h2),
        out_shape=jax.ShapeDtypeStruct((n, h2), jnp.float32),
    )(acc1, g1, dis, b1m, W2)

    acc2 = agg(g2, src, dst)

    out = pl.pallas_call(
        _tc3_body,
        grid=grid,
        in_specs=[_pair_spec(h2), _row_spec(h2), _row_spec(L),
                  _full_spec((1, h2))],
        out_specs=_row_spec(h2),
        out_shape=jax.ShapeDtypeStruct((n, h2), jnp.float32),
    )(acc2, g2, dis, b2m)

    return out
